# baseline (speedup 1.0000x reference)
import jax
import jax.numpy as jnp
from jax.experimental import pallas as pl
from jax.experimental.pallas import tpu as pltpu

_BM = 1024


def _mlp_block(x_ref, m_ref, w_ref, w1_ref, b1_ref, w2_ref, b2_ref, w3_ref,
               b3_ref, o_ref):
    f32 = jnp.float32
    wm = w_ref[:] * m_ref[:].astype(f32)[:, None]
    h = jnp.dot(x_ref[:], wm, preferred_element_type=f32)
    h = jnp.maximum(
        jnp.dot(h, w1_ref[:], preferred_element_type=f32) + b1_ref[:][None, :],
        0.0)
    h = jnp.maximum(
        jnp.dot(h, w2_ref[:], preferred_element_type=f32) + b2_ref[:][None, :],
        0.0)
    o_ref[:] = (jnp.dot(h, w3_ref[:], preferred_element_type=f32) +
                b3_ref[:][None, :])


def kernel(x, feature_mask, W, W1, b1, W2, b2, W3, b3):
    batch, feat = x.shape
    hidden = W.shape[1]
    classes = W3.shape[1]
    bm = min(_BM, batch)
    grid = (batch // bm,)
    full = lambda i: (0,)
    return pl.pallas_call(
        _mlp_block,
        grid=grid,
        compiler_params=pltpu.CompilerParams(
            dimension_semantics=("parallel",)),
        in_specs=[
            pl.BlockSpec((bm, feat), lambda i: (i, 0)),
            pl.BlockSpec((feat,), full),
            pl.BlockSpec((feat, hidden), lambda i: (0, 0)),
            pl.BlockSpec((hidden, hidden), lambda i: (0, 0)),
            pl.BlockSpec((hidden,), full),
            pl.BlockSpec((hidden, hidden), lambda i: (0, 0)),
            pl.BlockSpec((hidden,), full),
            pl.BlockSpec((hidden, classes), lambda i: (0, 0)),
            pl.BlockSpec((classes,), full),
        ],
        out_specs=pl.BlockSpec((bm, classes), lambda i: (i, 0)),
        out_shape=jax.ShapeDtypeStruct((batch, classes), x.dtype),
    )(x, feature_mask, W, W1, b1, W2, b2, W3, b3)
